# baseline (speedup 1.0000x reference)
import jax
import jax.numpy as jnp
from jax import lax
from jax.experimental import pallas as pl
from jax.experimental.pallas import tpu as pltpu
from jax.experimental.pallas import tpu_sc as plsc

N = 10000
E = 320000
D = 128
G = 64

NC = 2
NS = 16
NW = NC * NS

CHUNK = 128
NCHUNKS = E // CHUNK
CH_PER_TILE = -(-NCHUNKS // NW)

ROWS_MAIN = 624
ROWS_TAIL = N - NS * ROWS_MAIN
ZR = 104
SEG = D // 16


def _edge_pass_body(h_hbm, src_hbm, dst_hbm, attr_hbm, table_hbm, out_hbm,
                    src_v, dst_v, attr_v, rows_v, emb_v, zero_v, agg_sh):
    c = lax.axis_index("c")
    s = lax.axis_index("s")
    wid = c * NS + s

    @pl.loop(0, ZR)
    def _(r):
        for g in range(SEG):
            zero_v[r, pl.ds(g * 16, 16)] = jnp.zeros((16,), jnp.float32)

    base_r = s * ROWS_MAIN
    for b in range(ROWS_MAIN // ZR):
        pltpu.sync_copy(zero_v, agg_sh.at[pl.ds(base_r + b * ZR, ZR)])

    @pl.when(s == NS - 1)
    def _():
        pltpu.sync_copy(zero_v.at[pl.ds(0, ROWS_TAIL)],
                        agg_sh.at[pl.ds(NS * ROWS_MAIN, ROWS_TAIL)])

    plsc.subcore_barrier()

    @pl.loop(0, CH_PER_TILE)
    def _(j):
        cidx = j * NW + wid

        @pl.when(cidx < NCHUNKS)
        def _():
            base = cidx * CHUNK
            pltpu.sync_copy(src_hbm.at[pl.ds(base, CHUNK)], src_v)
            pltpu.sync_copy(dst_hbm.at[pl.ds(base, CHUNK)], dst_v)
            pltpu.sync_copy(attr_hbm.at[pl.ds(base, CHUNK)], attr_v)
            pltpu.sync_copy(h_hbm.at[src_v], rows_v)
            pltpu.sync_copy(table_hbm.at[attr_v], emb_v)

            @pl.loop(0, CHUNK)
            def _(e):
                for g in range(SEG):
                    sl = pl.ds(g * 16, 16)
                    rows_v[e, sl] = jnp.maximum(rows_v[e, sl] + emb_v[e, sl],
                                                0.0)

            pltpu.sync_copy(rows_v, agg_sh.at[dst_v], add=True)

    plsc.subcore_barrier()

    pltpu.sync_copy(agg_sh.at[pl.ds(base_r, ROWS_MAIN)],
                    out_hbm.at[c, pl.ds(base_r, ROWS_MAIN)])

    @pl.when(s == NS - 1)
    def _():
        pltpu.sync_copy(agg_sh.at[pl.ds(NS * ROWS_MAIN, ROWS_TAIL)],
                        out_hbm.at[c, pl.ds(NS * ROWS_MAIN, ROWS_TAIL)])


_edge_pass = pl.kernel(
    _edge_pass_body,
    mesh=plsc.VectorSubcoreMesh(core_axis_name="c", subcore_axis_name="s"),
    out_type=jax.ShapeDtypeStruct((NC, N, D), jnp.float32),
    scratch_types=[
        pltpu.VMEM((CHUNK,), jnp.int32),
        pltpu.VMEM((CHUNK,), jnp.int32),
        pltpu.VMEM((CHUNK,), jnp.int32),
        pltpu.VMEM((CHUNK, D), jnp.float32),
        pltpu.VMEM((CHUNK, D), jnp.float32),
        pltpu.VMEM((ZR, D), jnp.float32),
        pltpu.VMEM_SHARED((N, D), jnp.float32),
    ],
)


BLK = 400
NBLK = N // BLK


def _mlp_body(x_ref, agg_ref, w1_ref, b1_ref, w2_ref, b2_ref, o_ref):
    a = agg_ref[...]
    t = x_ref[...] + a[0] + a[1]
    z = jnp.maximum(
        jnp.dot(t, w1_ref[...], preferred_element_type=jnp.float32)
        + b1_ref[...], 0.0)
    o_ref[...] = (jnp.dot(z, w2_ref[...], preferred_element_type=jnp.float32)
                  + b2_ref[...])


def _mlp(x, agg, W1, b1, W2, b2):
    return pl.pallas_call(
        _mlp_body,
        grid=(NBLK,),
        in_specs=[
            pl.BlockSpec((BLK, D), lambda i: (i, 0)),
            pl.BlockSpec((NC, BLK, D), lambda i: (0, i, 0)),
            pl.BlockSpec((D, D), lambda i: (0, 0)),
            pl.BlockSpec((1, D), lambda i: (0, 0)),
            pl.BlockSpec((D, D), lambda i: (0, 0)),
            pl.BlockSpec((1, D), lambda i: (0, 0)),
        ],
        out_specs=pl.BlockSpec((BLK, D), lambda i: (i, 0)),
        out_shape=jax.ShapeDtypeStruct((N, D), jnp.float32),
    )(x, agg, W1, b1.reshape(1, D), W2, b2.reshape(1, D))


def _mlp_pool_body(x_ref, agg_ref, batch_ref, w1_ref, b1_ref, w2_ref, b2_ref,
                   o_ref, sum_scr, cnt_scr):
    i = pl.program_id(0)
    a = agg_ref[...]
    t = x_ref[...] + a[0] + a[1]
    z = jnp.maximum(
        jnp.dot(t, w1_ref[...], preferred_element_type=jnp.float32)
        + b1_ref[...], 0.0)
    h2 = (jnp.dot(z, w2_ref[...], preferred_element_type=jnp.float32)
          + b2_ref[...])

    bids = batch_ref[0, 0, :]
    oh = (lax.broadcasted_iota(jnp.int32, (G, BLK), 0)
          == bids[None, :]).astype(jnp.float32)
    psum = jnp.dot(oh, h2, preferred_element_type=jnp.float32)
    pcnt = jnp.sum(oh, axis=1)

    @pl.when(i == 0)
    def _():
        sum_scr[...] = jnp.zeros_like(sum_scr)
        cnt_scr[...] = jnp.zeros_like(cnt_scr)

    sum_scr[...] += psum
    cnt_scr[...] += pcnt[:, None]

    @pl.when(i == NBLK - 1)
    def _():
        o_ref[...] = sum_scr[...] / jnp.maximum(cnt_scr[...], 1.0)


def _mlp_pool(x, agg, batch_r, W1, b1, W2, b2):
    return pl.pallas_call(
        _mlp_pool_body,
        grid=(NBLK,),
        in_specs=[
            pl.BlockSpec((BLK, D), lambda i: (i, 0)),
            pl.BlockSpec((NC, BLK, D), lambda i: (0, i, 0)),
            pl.BlockSpec((1, 1, BLK), lambda i: (i, 0, 0)),
            pl.BlockSpec((D, D), lambda i: (0, 0)),
            pl.BlockSpec((1, D), lambda i: (0, 0)),
            pl.BlockSpec((D, D), lambda i: (0, 0)),
            pl.BlockSpec((1, D), lambda i: (0, 0)),
        ],
        out_specs=pl.BlockSpec((G, D), lambda i: (0, 0)),
        out_shape=jax.ShapeDtypeStruct((G, D), jnp.float32),
        scratch_shapes=[
            pltpu.VMEM((G, D), jnp.float32),
            pltpu.VMEM((G, D), jnp.float32),
        ],
    )(x, agg, batch_r, W1, b1.reshape(1, D), W2, b2.reshape(1, D))


def kernel(x, edge_index, edge_attr, batch, edge_table,
           W11, b11, W12, b12, W21, b21, W22, b22):
    src = edge_index[0]
    dst = edge_index[1]
    batch_r = batch.reshape(NBLK, 1, BLK)

    agg1 = _edge_pass(x, src, dst, edge_attr, edge_table)
    h1 = _mlp(x, agg1, W11, b11, W12, b12)
    agg2 = _edge_pass(h1, src, dst, edge_attr, edge_table)
    return _mlp_pool(h1, agg2, batch_r, W21, b21, W22, b22)


# speedup vs baseline: 14.6616x; 14.6616x over previous
import jax
import jax.numpy as jnp
from jax import lax
from jax.experimental import pallas as pl
from jax.experimental.pallas import tpu as pltpu
from jax.experimental.pallas import tpu_sc as plsc

N = 10000
E = 320000
D = 128
G = 64

NC = 2
NS = 16
NW = NC * NS

CHUNK = 16
CPT = E // (NW * CHUNK)
NBUF = 5
LOOK = 2

ROWS_MAIN = 624
ROWS_TAIL = N - NS * ROWS_MAIN


EPT = E // NW


def _edge_pass_body(h4_hbm, cidx_hbm, dst_hbm, out_hbm,
                    cidx_v, dst_v, rows_v, agg_sh, gsem, ssem):
    c = lax.axis_index("c")
    s = lax.axis_index("s")
    wid = c * NS + s

    cbase = wid * EPT
    pltpu.sync_copy(cidx_hbm.at[pl.ds(cbase, EPT)], cidx_v)
    pltpu.sync_copy(dst_hbm.at[pl.ds(cbase, EPT)], dst_v)

    def cidx_at(j):
        return cidx_v[pl.ds(pl.multiple_of(j * CHUNK, CHUNK), CHUNK)]

    def dst_at(j):
        return dst_v[pl.ds(pl.multiple_of(j * CHUNK, CHUNK), CHUNK)]

    @pl.loop(0, CHUNK)
    def _(r):
        for g in range(D // 16):
            rows_v[0][r, pl.ds(g * 16, 16)] = jnp.zeros((16,), jnp.float32)

    base_r = s * ROWS_MAIN
    for b in range(ROWS_MAIN // CHUNK):
        pltpu.sync_copy(rows_v[0], agg_sh.at[pl.ds(base_r + b * CHUNK, CHUNK)])

    @pl.when(s == NS - 1)
    def _():
        pltpu.sync_copy(rows_v[0], agg_sh.at[pl.ds(NS * ROWS_MAIN, ROWS_TAIL)])

    plsc.subcore_barrier()

    for jp in range(LOOK):
        pltpu.async_copy(h4_hbm.at[cidx_at(jp)], rows_v[jp], gsem[jp])

    @pl.loop(0, CPT // NBUF)
    def _(j0):
        for d in range(NBUF):
            j = j0 * NBUF + d
            bn = (d + LOOK) % NBUF
            jn = j + LOOK

            @pl.when(j >= NBUF - LOOK)
            def _():
                pltpu.make_async_copy(
                    rows_v[bn], agg_sh.at[dst_at(j - (NBUF - LOOK))],
                    ssem[bn]).wait()

            @pl.when(jn < CPT)
            def _():
                pltpu.async_copy(h4_hbm.at[cidx_at(jn)], rows_v[bn],
                                 gsem[bn])

            pltpu.make_async_copy(h4_hbm.at[cidx_at(j)], rows_v[d],
                                  gsem[d]).wait()

            pltpu.async_copy(rows_v[d], agg_sh.at[dst_at(j)], ssem[d],
                             add=True)

    for t in range(LOOK, NBUF):
        pltpu.make_async_copy(rows_v[t], agg_sh.at[dst_at(CPT - NBUF + t)],
                              ssem[t]).wait()

    plsc.subcore_barrier()

    pltpu.sync_copy(agg_sh.at[pl.ds(base_r, ROWS_MAIN)],
                    out_hbm.at[c, pl.ds(base_r, ROWS_MAIN)])

    @pl.when(s == NS - 1)
    def _():
        pltpu.sync_copy(agg_sh.at[pl.ds(NS * ROWS_MAIN, ROWS_TAIL)],
                        out_hbm.at[c, pl.ds(NS * ROWS_MAIN, ROWS_TAIL)])


_edge_pass = pl.kernel(
    _edge_pass_body,
    mesh=plsc.VectorSubcoreMesh(core_axis_name="c", subcore_axis_name="s"),
    out_type=jax.ShapeDtypeStruct((NC, N, D), jnp.float32),
    scratch_types=[
        pltpu.VMEM((E // NW,), jnp.int32),
        pltpu.VMEM((E // NW,), jnp.int32),
        [pltpu.VMEM((CHUNK, D), jnp.float32) for _ in range(NBUF)],
        pltpu.VMEM_SHARED((N, D), jnp.float32),
        [pltpu.SemaphoreType.DMA for _ in range(NBUF)],
        [pltpu.SemaphoreType.DMA for _ in range(NBUF)],
    ],
)


BLK = 400
NBLK = N // BLK


def _h4_body(x_ref, t_ref, o_ref):
    xv = x_ref[...]
    tv = t_ref[...]
    for a in range(4):
        o_ref[a] = jnp.maximum(xv + tv[a], 0.0)


def _h4(x, table):
    return pl.pallas_call(
        _h4_body,
        grid=(NBLK,),
        in_specs=[
            pl.BlockSpec((BLK, D), lambda i: (i, 0)),
            pl.BlockSpec((4, D), lambda i: (0, 0)),
        ],
        out_specs=pl.BlockSpec((4, BLK, D), lambda i: (0, i, 0)),
        out_shape=jax.ShapeDtypeStruct((4, N, D), jnp.float32),
    )(x, table)


def _mlp_h4_body(x_ref, agg_ref, w1_ref, b1_ref, w2_ref, b2_ref, t_ref,
                 h_ref, h4_ref):
    a = agg_ref[...]
    t = x_ref[...] + a[0] + a[1]
    z = jnp.maximum(
        jnp.dot(t, w1_ref[...], preferred_element_type=jnp.float32)
        + b1_ref[...], 0.0)
    h = (jnp.dot(z, w2_ref[...], preferred_element_type=jnp.float32)
         + b2_ref[...])
    h_ref[...] = h
    tv = t_ref[...]
    for k in range(4):
        h4_ref[k] = jnp.maximum(h + tv[k], 0.0)


def _mlp_h4(x, agg, W1, b1, W2, b2, table):
    return pl.pallas_call(
        _mlp_h4_body,
        grid=(NBLK,),
        in_specs=[
            pl.BlockSpec((BLK, D), lambda i: (i, 0)),
            pl.BlockSpec((NC, BLK, D), lambda i: (0, i, 0)),
            pl.BlockSpec((D, D), lambda i: (0, 0)),
            pl.BlockSpec((1, D), lambda i: (0, 0)),
            pl.BlockSpec((D, D), lambda i: (0, 0)),
            pl.BlockSpec((1, D), lambda i: (0, 0)),
            pl.BlockSpec((4, D), lambda i: (0, 0)),
        ],
        out_specs=[
            pl.BlockSpec((BLK, D), lambda i: (i, 0)),
            pl.BlockSpec((4, BLK, D), lambda i: (0, i, 0)),
        ],
        out_shape=[
            jax.ShapeDtypeStruct((N, D), jnp.float32),
            jax.ShapeDtypeStruct((4, N, D), jnp.float32),
        ],
    )(x, agg, W1, b1.reshape(1, D), W2, b2.reshape(1, D), table)


def _mlp_pool_body(x_ref, agg_ref, batch_ref, w1_ref, b1_ref, w2_ref, b2_ref,
                   o_ref, sum_scr, cnt_scr):
    i = pl.program_id(0)
    a = agg_ref[...]
    t = x_ref[...] + a[0] + a[1]
    z = jnp.maximum(
        jnp.dot(t, w1_ref[...], preferred_element_type=jnp.float32)
        + b1_ref[...], 0.0)
    h2 = (jnp.dot(z, w2_ref[...], preferred_element_type=jnp.float32)
          + b2_ref[...])

    bids = batch_ref[0, 0, :]
    oh = (lax.broadcasted_iota(jnp.int32, (G, BLK), 0)
          == bids[None, :]).astype(jnp.float32)
    psum = jnp.dot(oh, h2, preferred_element_type=jnp.float32)
    pcnt = jnp.sum(oh, axis=1)

    @pl.when(i == 0)
    def _():
        sum_scr[...] = jnp.zeros_like(sum_scr)
        cnt_scr[...] = jnp.zeros_like(cnt_scr)

    sum_scr[...] += psum
    cnt_scr[...] += pcnt[:, None]

    @pl.when(i == NBLK - 1)
    def _():
        o_ref[...] = sum_scr[...] / jnp.maximum(cnt_scr[...], 1.0)


def _mlp_pool(x, agg, batch_r, W1, b1, W2, b2):
    return pl.pallas_call(
        _mlp_pool_body,
        grid=(NBLK,),
        in_specs=[
            pl.BlockSpec((BLK, D), lambda i: (i, 0)),
            pl.BlockSpec((NC, BLK, D), lambda i: (0, i, 0)),
            pl.BlockSpec((1, 1, BLK), lambda i: (i, 0, 0)),
            pl.BlockSpec((D, D), lambda i: (0, 0)),
            pl.BlockSpec((1, D), lambda i: (0, 0)),
            pl.BlockSpec((D, D), lambda i: (0, 0)),
            pl.BlockSpec((1, D), lambda i: (0, 0)),
        ],
        out_specs=pl.BlockSpec((G, D), lambda i: (0, 0)),
        out_shape=jax.ShapeDtypeStruct((G, D), jnp.float32),
        scratch_shapes=[
            pltpu.VMEM((G, D), jnp.float32),
            pltpu.VMEM((G, D), jnp.float32),
        ],
    )(x, agg, batch_r, W1, b1.reshape(1, D), W2, b2.reshape(1, D))


def kernel(x, edge_index, edge_attr, batch, edge_table,
           W11, b11, W12, b12, W21, b21, W22, b22):
    cidx = edge_attr * N + edge_index[0]
    dst = edge_index[1]
    batch_r = batch.reshape(NBLK, 1, BLK)

    h4a = _h4(x, edge_table).reshape(4 * N, D)
    agg1 = _edge_pass(h4a, cidx, dst)
    h1, h4b = _mlp_h4(x, agg1, W11, b11, W12, b12, edge_table)
    agg2 = _edge_pass(h4b.reshape(4 * N, D), cidx, dst)
    return _mlp_pool(h1, agg2, batch_r, W21, b21, W22, b22)


# speedup vs baseline: 17.4149x; 1.1878x over previous
import jax
import jax.numpy as jnp
from jax import lax
from jax.experimental import pallas as pl
from jax.experimental.pallas import tpu as pltpu
from jax.experimental.pallas import tpu_sc as plsc

N = 10000
E = 320000
D = 128
G = 64

NC = 2
NS = 16
NW = NC * NS

CHUNK = 16
CPT = E // (NW * CHUNK)
NBUF = 5
LOOK = 3

ROWS_MAIN = 624
ROWS_TAIL = N - NS * ROWS_MAIN


EPT = E // NW


def _edge_pass_body(h4_hbm, cidx_hbm, dst_hbm, out_hbm,
                    cidx_v, dst_v, rows_v, agg_sh, gsem, ssem):
    c = lax.axis_index("c")
    s = lax.axis_index("s")
    wid = c * NS + s

    cbase = wid * EPT
    pltpu.sync_copy(cidx_hbm.at[pl.ds(cbase, EPT)], cidx_v)
    pltpu.sync_copy(dst_hbm.at[pl.ds(cbase, EPT)], dst_v)

    def cidx_at(j):
        return cidx_v[pl.ds(pl.multiple_of(j * CHUNK, CHUNK), CHUNK)]

    def dst_at(j):
        return dst_v[pl.ds(pl.multiple_of(j * CHUNK, CHUNK), CHUNK)]

    for zb in (NBUF - 2, NBUF - 1):
        @pl.loop(0, CHUNK)
        def _(r):
            for g in range(D // 16):
                rows_v[zb][r, pl.ds(g * 16, 16)] = jnp.zeros((16,),
                                                             jnp.float32)

    base_r = s * ROWS_MAIN
    NZ = ROWS_MAIN // CHUNK
    for b in range(NZ):
        zb = NBUF - 2 + (b % 2)
        pltpu.async_copy(rows_v[zb],
                         agg_sh.at[pl.ds(base_r + b * CHUNK, CHUNK)],
                         ssem[zb])

    @pl.when(s == NS - 1)
    def _():
        pltpu.async_copy(rows_v[NBUF - 1],
                         agg_sh.at[pl.ds(NS * ROWS_MAIN, ROWS_TAIL)],
                         ssem[NBUF - 1])

    for jp in range(LOOK):
        pltpu.async_copy(h4_hbm.at[cidx_at(jp)], rows_v[jp], gsem[jp])

    for b in range(NZ):
        zb = NBUF - 2 + (b % 2)
        pltpu.make_async_copy(rows_v[zb],
                              agg_sh.at[pl.ds(base_r + b * CHUNK, CHUNK)],
                              ssem[zb]).wait()

    @pl.when(s == NS - 1)
    def _():
        pltpu.make_async_copy(rows_v[NBUF - 1],
                              agg_sh.at[pl.ds(NS * ROWS_MAIN, ROWS_TAIL)],
                              ssem[NBUF - 1]).wait()

    plsc.subcore_barrier()

    @pl.loop(0, CPT // NBUF)
    def _(j0):
        for d in range(NBUF):
            j = j0 * NBUF + d
            bn = (d + LOOK) % NBUF
            jn = j + LOOK

            @pl.when(j >= NBUF - LOOK)
            def _():
                pltpu.make_async_copy(
                    rows_v[bn], agg_sh.at[dst_at(j - (NBUF - LOOK))],
                    ssem[bn]).wait()

            @pl.when(jn < CPT)
            def _():
                pltpu.async_copy(h4_hbm.at[cidx_at(jn)], rows_v[bn],
                                 gsem[bn])

            pltpu.make_async_copy(h4_hbm.at[cidx_at(j)], rows_v[d],
                                  gsem[d]).wait()

            pltpu.async_copy(rows_v[d], agg_sh.at[dst_at(j)], ssem[d],
                             add=True)

    for t in range(LOOK, NBUF):
        pltpu.make_async_copy(rows_v[t], agg_sh.at[dst_at(CPT - NBUF + t)],
                              ssem[t]).wait()

    plsc.subcore_barrier()

    pltpu.sync_copy(agg_sh.at[pl.ds(base_r, ROWS_MAIN)],
                    out_hbm.at[c, pl.ds(base_r, ROWS_MAIN)])

    @pl.when(s == NS - 1)
    def _():
        pltpu.sync_copy(agg_sh.at[pl.ds(NS * ROWS_MAIN, ROWS_TAIL)],
                        out_hbm.at[c, pl.ds(NS * ROWS_MAIN, ROWS_TAIL)])


_edge_pass = pl.kernel(
    _edge_pass_body,
    mesh=plsc.VectorSubcoreMesh(core_axis_name="c", subcore_axis_name="s"),
    out_type=jax.ShapeDtypeStruct((NC, N, D), jnp.float32),
    scratch_types=[
        pltpu.VMEM((E // NW,), jnp.int32),
        pltpu.VMEM((E // NW,), jnp.int32),
        [pltpu.VMEM((CHUNK, D), jnp.float32) for _ in range(NBUF)],
        pltpu.VMEM_SHARED((N, D), jnp.float32),
        [pltpu.SemaphoreType.DMA for _ in range(NBUF)],
        [pltpu.SemaphoreType.DMA for _ in range(NBUF)],
    ],
)


BLK = 400
NBLK = N // BLK


def _h4_body(x_ref, t_ref, o_ref):
    xv = x_ref[...]
    tv = t_ref[...]
    for a in range(4):
        o_ref[a] = jnp.maximum(xv + tv[a], 0.0)


def _h4(x, table):
    return pl.pallas_call(
        _h4_body,
        grid=(NBLK,),
        in_specs=[
            pl.BlockSpec((BLK, D), lambda i: (i, 0)),
            pl.BlockSpec((4, D), lambda i: (0, 0)),
        ],
        out_specs=pl.BlockSpec((4, BLK, D), lambda i: (0, i, 0)),
        out_shape=jax.ShapeDtypeStruct((4, N, D), jnp.float32),
    )(x, table)


def _mlp_h4_body(x_ref, agg_ref, w1_ref, b1_ref, w2_ref, b2_ref, t_ref,
                 h_ref, h4_ref):
    a = agg_ref[...]
    t = x_ref[...] + a[0] + a[1]
    z = jnp.maximum(
        jnp.dot(t, w1_ref[...], preferred_element_type=jnp.float32)
        + b1_ref[...], 0.0)
    h = (jnp.dot(z, w2_ref[...], preferred_element_type=jnp.float32)
         + b2_ref[...])
    h_ref[...] = h
    tv = t_ref[...]
    for k in range(4):
        h4_ref[k] = jnp.maximum(h + tv[k], 0.0)


def _mlp_h4(x, agg, W1, b1, W2, b2, table):
    return pl.pallas_call(
        _mlp_h4_body,
        grid=(NBLK,),
        in_specs=[
            pl.BlockSpec((BLK, D), lambda i: (i, 0)),
            pl.BlockSpec((NC, BLK, D), lambda i: (0, i, 0)),
            pl.BlockSpec((D, D), lambda i: (0, 0)),
            pl.BlockSpec((1, D), lambda i: (0, 0)),
            pl.BlockSpec((D, D), lambda i: (0, 0)),
            pl.BlockSpec((1, D), lambda i: (0, 0)),
            pl.BlockSpec((4, D), lambda i: (0, 0)),
        ],
        out_specs=[
            pl.BlockSpec((BLK, D), lambda i: (i, 0)),
            pl.BlockSpec((4, BLK, D), lambda i: (0, i, 0)),
        ],
        out_shape=[
            jax.ShapeDtypeStruct((N, D), jnp.float32),
            jax.ShapeDtypeStruct((4, N, D), jnp.float32),
        ],
    )(x, agg, W1, b1.reshape(1, D), W2, b2.reshape(1, D), table)


def _mlp_pool_body(x_ref, agg_ref, batch_ref, w1_ref, b1_ref, w2_ref, b2_ref,
                   o_ref, sum_scr, cnt_scr):
    i = pl.program_id(0)
    a = agg_ref[...]
    t = x_ref[...] + a[0] + a[1]
    z = jnp.maximum(
        jnp.dot(t, w1_ref[...], preferred_element_type=jnp.float32)
        + b1_ref[...], 0.0)
    h2 = (jnp.dot(z, w2_ref[...], preferred_element_type=jnp.float32)
          + b2_ref[...])

    bids = batch_ref[0, 0, :]
    oh = (lax.broadcasted_iota(jnp.int32, (G, BLK), 0)
          == bids[None, :]).astype(jnp.float32)
    psum = jnp.dot(oh, h2, preferred_element_type=jnp.float32)
    pcnt = jnp.sum(oh, axis=1)

    @pl.when(i == 0)
    def _():
        sum_scr[...] = jnp.zeros_like(sum_scr)
        cnt_scr[...] = jnp.zeros_like(cnt_scr)

    sum_scr[...] += psum
    cnt_scr[...] += pcnt[:, None]

    @pl.when(i == NBLK - 1)
    def _():
        o_ref[...] = sum_scr[...] / jnp.maximum(cnt_scr[...], 1.0)


def _mlp_pool(x, agg, batch_r, W1, b1, W2, b2):
    return pl.pallas_call(
        _mlp_pool_body,
        grid=(NBLK,),
        in_specs=[
            pl.BlockSpec((BLK, D), lambda i: (i, 0)),
            pl.BlockSpec((NC, BLK, D), lambda i: (0, i, 0)),
            pl.BlockSpec((1, 1, BLK), lambda i: (i, 0, 0)),
            pl.BlockSpec((D, D), lambda i: (0, 0)),
            pl.BlockSpec((1, D), lambda i: (0, 0)),
            pl.BlockSpec((D, D), lambda i: (0, 0)),
            pl.BlockSpec((1, D), lambda i: (0, 0)),
        ],
        out_specs=pl.BlockSpec((G, D), lambda i: (0, 0)),
        out_shape=jax.ShapeDtypeStruct((G, D), jnp.float32),
        scratch_shapes=[
            pltpu.VMEM((G, D), jnp.float32),
            pltpu.VMEM((G, D), jnp.float32),
        ],
    )(x, agg, batch_r, W1, b1.reshape(1, D), W2, b2.reshape(1, D))


def kernel(x, edge_index, edge_attr, batch, edge_table,
           W11, b11, W12, b12, W21, b21, W22, b22):
    cidx = edge_attr * N + edge_index[0]
    dst = edge_index[1]
    batch_r = batch.reshape(NBLK, 1, BLK)

    h4a = _h4(x, edge_table).reshape(4 * N, D)
    agg1 = _edge_pass(h4a, cidx, dst)
    h1, h4b = _mlp_h4(x, agg1, W11, b11, W12, b12, edge_table)
    agg2 = _edge_pass(h4b.reshape(4 * N, D), cidx, dst)
    return _mlp_pool(h1, agg2, batch_r, W21, b21, W22, b22)
